# Initial kernel scaffold; baseline (speedup 1.0000x reference)
import jax
import jax.numpy as jnp
from jax.experimental import pallas as pl


def kernel(x, pos, batch, params):
    raise NotImplementedError("write your pallas kernel here")


# speedup vs baseline: 1.1411x; 1.1411x over previous
import functools

import jax
import jax.numpy as jnp
import numpy as np
from jax.experimental import pallas as pl
from jax.experimental.pallas import tpu as pltpu

_BN_SCALE = 1.0 / np.sqrt(1.0 + 1e-5)


def _mlp_body(nlayers, h_ref, *refs):
    o_ref = refs[-1]
    params = refs[:-1]
    h = h_ref[...]
    for li in range(nlayers):
        W, b, g, be = params[4 * li: 4 * li + 4]
        h = jax.lax.dot_general(h, W[...], (((1,), (0,)), ((), ())),
                                preferred_element_type=jnp.float32)
        h = h + b[...]
        h = jnp.maximum(h, 0.0)
        h = g[...] * (h * _BN_SCALE) + be[...]
    o_ref[...] = h


def _mlp_pallas(h, layers, block_rows):
    rows, cin = h.shape
    assert rows % block_rows == 0, (rows, block_rows)
    grid = rows // block_rows
    nlayers = len(layers)
    cout = layers[-1][0].shape[1]
    flat = []
    in_specs = [pl.BlockSpec((block_rows, cin), lambda i: (i, 0))]
    for (W, b, g, be) in layers:
        for arr2 in (W, b.reshape(1, -1), g.reshape(1, -1), be.reshape(1, -1)):
            flat.append(arr2)
            in_specs.append(pl.BlockSpec(arr2.shape, lambda i: (0, 0)))
    return pl.pallas_call(
        functools.partial(_mlp_body, nlayers),
        grid=(grid,),
        in_specs=in_specs,
        out_specs=pl.BlockSpec((block_rows, cout), lambda i: (i, 0)),
        out_shape=jax.ShapeDtypeStruct((rows, cout), jnp.float32),
    )(h, *flat)


def _fps(pos, num):
    p = pos
    N = p.shape[0]
    idxs = jnp.zeros((num,), jnp.int32)
    dists = jnp.full((N,), jnp.inf, jnp.float32)

    def body(i, carry):
        idxs, dists = carry
        last = p[idxs[i - 1]]
        d = jnp.sum((p - last[None, :]) ** 2, axis=1)
        dists = jnp.minimum(dists, d)
        nxt = jnp.argmax(dists).astype(jnp.int32)
        return idxs.at[i].set(nxt), dists

    idxs, _ = jax.lax.fori_loop(1, num, body, (idxs, dists))
    return idxs


def _sa(x, pos, num_centroids, r2, layers, K=64):
    fps_idx = _fps(pos, num_centroids)
    pos_q = pos[fps_idx]
    d2 = jnp.sum((pos_q[:, None, :] - pos[None, :, :]) ** 2, axis=-1)
    neg_d2, nbr = jax.lax.top_k(-d2, K)
    valid = (-neg_d2) <= r2
    x_j = x[nbr]
    rel = pos[nbr] - pos_q[:, None, :]
    h = jnp.concatenate([x_j, rel], axis=-1)
    S, _, C = h.shape
    block = 2560 if S * K == 128000 else 2000
    hout = _mlp_pallas(h.reshape(S * K, C), layers, block)
    hout = hout.reshape(S, K, -1)
    hout = jnp.where(valid[:, :, None], hout, -1e10)
    return jnp.max(hout, axis=1), pos_q, fps_idx


def kernel(x, pos, batch, params):
    N = x.shape[0]
    s1 = int(N * 0.2)
    x1, pos1, idx1 = _sa(x, pos, s1, np.float32(0.2 * 0.2), params["mlp1"])
    b1 = batch[idx1]
    s2 = int(s1 * 0.25)
    x2, pos2, idx2 = _sa(x1, pos1, s2, np.float32(0.4 * 0.4), params["mlp2"])
    b2 = b1[idx2]
    h = jnp.concatenate([x2, pos2], axis=-1)
    hp = jnp.pad(h, ((0, 12), (0, 0)))
    hout = _mlp_pallas(hp, params["mlp3"], 512)[:500]
    x3 = jnp.max(hout, axis=0, keepdims=True)
    pos3 = jnp.zeros((1, 3), x.dtype)
    b3 = jnp.arange(1, dtype=jnp.int32)
    return ((x, pos, batch), (x1, pos1, b1), (x2, pos2, b2), (x3, pos3, b3))


# speedup vs baseline: 3.0336x; 2.6586x over previous
import functools

import jax
import jax.numpy as jnp
import numpy as np
from jax.experimental import pallas as pl
from jax.experimental.pallas import tpu as pltpu

_BN_SCALE = 1.0 / np.sqrt(1.0 + 1e-5)


def _mlp_body(nlayers, h_ref, *refs):
    o_ref = refs[-1]
    params = refs[:-1]
    h = h_ref[...]
    for li in range(nlayers):
        W, b, g, be = params[4 * li: 4 * li + 4]
        h = jax.lax.dot_general(h, W[...], (((1,), (0,)), ((), ())),
                                preferred_element_type=jnp.float32)
        h = h + b[...]
        h = jnp.maximum(h, 0.0)
        h = g[...] * (h * _BN_SCALE) + be[...]
    o_ref[...] = h


def _mlp_pallas(h, layers, block_rows):
    rows, cin = h.shape
    assert rows % block_rows == 0, (rows, block_rows)
    grid = rows // block_rows
    nlayers = len(layers)
    cout = layers[-1][0].shape[1]
    flat = []
    in_specs = [pl.BlockSpec((block_rows, cin), lambda i: (i, 0))]
    for (W, b, g, be) in layers:
        for arr2 in (W, b.reshape(1, -1), g.reshape(1, -1), be.reshape(1, -1)):
            flat.append(arr2)
            in_specs.append(pl.BlockSpec(arr2.shape, lambda i: (0, 0)))
    return pl.pallas_call(
        functools.partial(_mlp_body, nlayers),
        grid=(grid,),
        in_specs=in_specs,
        out_specs=pl.BlockSpec((block_rows, cout), lambda i: (i, 0)),
        out_shape=jax.ShapeDtypeStruct((rows, cout), jnp.float32),
    )(h, *flat)


def _fps_body(num, n_real, px_ref, py_ref, pz_ref, idx_ref, dists_ref):
    rows, cols = px_ref.shape
    flat = (jax.lax.broadcasted_iota(jnp.int32, (rows, cols), 0) * cols
            + jax.lax.broadcasted_iota(jnp.int32, (rows, cols), 1))
    real = flat < n_real
    px, py, pz = px_ref[...], py_ref[...], pz_ref[...]

    def extract(sel, arr):
        return jnp.sum(jnp.where(sel, arr, 0.0))

    sel0 = flat == 0
    ax, ay, az = extract(sel0, px), extract(sel0, py), extract(sel0, pz)
    idx_ref[0] = jnp.int32(0)
    dists_ref[...] = jnp.where(real, jnp.inf, -jnp.inf)

    def body(i, carry):
        ax, ay, az = carry
        d = (px - ax) ** 2 + (py - ay) ** 2 + (pz - az) ** 2
        dists = jnp.minimum(dists_ref[...], d)
        dists_ref[...] = dists
        m = jnp.max(dists)
        cand = jnp.where(dists == m, flat, jnp.int32(2 ** 30))
        nxt = jnp.min(cand)
        idx_ref[i] = nxt
        sel = flat == nxt
        return extract(sel, px), extract(sel, py), extract(sel, pz)

    jax.lax.fori_loop(1, num, body, (ax, ay, az), unroll=False)


def _fps(pos, num):
    N = pos.shape[0]
    cols = 128
    rows = (N + cols - 1) // cols
    rows = ((rows + 7) // 8) * 8
    npad = rows * cols - N
    p = jnp.pad(pos, ((0, npad), (0, 0)))
    px = p[:, 0].reshape(rows, cols)
    py = p[:, 1].reshape(rows, cols)
    pz = p[:, 2].reshape(rows, cols)
    return pl.pallas_call(
        functools.partial(_fps_body, num, N),
        in_specs=[pl.BlockSpec((rows, cols), lambda: (0, 0))] * 3,
        out_specs=pl.BlockSpec(memory_space=pltpu.SMEM),
        out_shape=jax.ShapeDtypeStruct((num,), jnp.int32),
        scratch_shapes=[pltpu.VMEM((rows, cols), jnp.float32)],
    )(px, py, pz)


def _sa(x, pos, num_centroids, r2, layers, K=64):
    fps_idx = _fps(pos, num_centroids)
    pos_q = pos[fps_idx]
    d2 = jnp.sum((pos_q[:, None, :] - pos[None, :, :]) ** 2, axis=-1)
    neg_d2, nbr = jax.lax.top_k(-d2, K)
    valid = (-neg_d2) <= r2
    x_j = x[nbr]
    rel = pos[nbr] - pos_q[:, None, :]
    h = jnp.concatenate([x_j, rel], axis=-1)
    S, _, C = h.shape
    block = 2560 if S * K == 128000 else 2000
    hout = _mlp_pallas(h.reshape(S * K, C), layers, block)
    hout = hout.reshape(S, K, -1)
    hout = jnp.where(valid[:, :, None], hout, -1e10)
    return jnp.max(hout, axis=1), pos_q, fps_idx


def kernel(x, pos, batch, params):
    N = x.shape[0]
    s1 = int(N * 0.2)
    x1, pos1, idx1 = _sa(x, pos, s1, np.float32(0.2 * 0.2), params["mlp1"])
    b1 = batch[idx1]
    s2 = int(s1 * 0.25)
    x2, pos2, idx2 = _sa(x1, pos1, s2, np.float32(0.4 * 0.4), params["mlp2"])
    b2 = b1[idx2]
    h = jnp.concatenate([x2, pos2], axis=-1)
    hp = jnp.pad(h, ((0, 12), (0, 0)))
    hout = _mlp_pallas(hp, params["mlp3"], 512)[:500]
    x3 = jnp.max(hout, axis=0, keepdims=True)
    pos3 = jnp.zeros((1, 3), x.dtype)
    b3 = jnp.arange(1, dtype=jnp.int32)
    return ((x, pos, batch), (x1, pos1, b1), (x2, pos2, b2), (x3, pos3, b3))


# speedup vs baseline: 3.1708x; 1.0452x over previous
import functools

import jax
import jax.numpy as jnp
import numpy as np
from jax.experimental import pallas as pl
from jax.experimental.pallas import tpu as pltpu

_BN_SCALE = 1.0 / np.sqrt(1.0 + 1e-5)


def _mlp_body(nlayers, h_ref, *refs):
    o_ref = refs[-1]
    params = refs[:-1]
    h = h_ref[...]
    for li in range(nlayers):
        W, b, g, be = params[4 * li: 4 * li + 4]
        h = jax.lax.dot_general(h, W[...], (((1,), (0,)), ((), ())),
                                preferred_element_type=jnp.float32)
        h = h + b[...]
        h = jnp.maximum(h, 0.0)
        h = g[...] * (h * _BN_SCALE) + be[...]
    o_ref[...] = h


def _mlp_pallas(h, layers, block_rows):
    rows, cin = h.shape
    assert rows % block_rows == 0, (rows, block_rows)
    grid = rows // block_rows
    nlayers = len(layers)
    cout = layers[-1][0].shape[1]
    flat = []
    in_specs = [pl.BlockSpec((block_rows, cin), lambda i: (i, 0))]
    for (W, b, g, be) in layers:
        for arr2 in (W, b.reshape(1, -1), g.reshape(1, -1), be.reshape(1, -1)):
            flat.append(arr2)
            in_specs.append(pl.BlockSpec(arr2.shape, lambda i: (0, 0)))
    return pl.pallas_call(
        functools.partial(_mlp_body, nlayers),
        grid=(grid,),
        in_specs=in_specs,
        out_specs=pl.BlockSpec((block_rows, cout), lambda i: (i, 0)),
        out_shape=jax.ShapeDtypeStruct((rows, cout), jnp.float32),
    )(h, *flat)


def _sa_mlp_body(nlayers, cx, K, r2, g_ref, q_ref, *refs):
    o_ref = refs[-1]
    params = refs[:-1]
    Sb = q_ref.shape[0]
    xj = g_ref[:, :cx]
    pj = g_ref[:, cx:cx + 3]
    d2col = g_ref[:, cx + 3:cx + 4]
    rel = (pj.reshape(Sb, K, 3) - q_ref[...][:, None, :]).reshape(Sb * K, 3)
    h = jnp.concatenate([xj, rel], axis=-1)
    for li in range(nlayers):
        W, b, g, be = params[4 * li: 4 * li + 4]
        h = jax.lax.dot_general(h, W[...], (((1,), (0,)), ((), ())),
                                preferred_element_type=jnp.float32)
        h = h + b[...]
        h = jnp.maximum(h, 0.0)
        h = g[...] * (h * _BN_SCALE) + be[...]
    cout = h.shape[-1]
    h = jnp.where(d2col <= r2, h, -1e10)
    o_ref[...] = jnp.max(h.reshape(Sb, K, cout), axis=1)


def _sa_mlp(gath, pos_q, layers, r2, Sb, K=64):
    Sp = pos_q.shape[0]
    Dg = gath.shape[1]
    cx = layers[0][0].shape[0] - 3
    nlayers = len(layers)
    cout = layers[-1][0].shape[1]
    flat = []
    in_specs = [
        pl.BlockSpec((Sb * K, Dg), lambda i: (i, 0)),
        pl.BlockSpec((Sb, 3), lambda i: (i, 0)),
    ]
    for (W, b, g, be) in layers:
        for arr2 in (W, b.reshape(1, -1), g.reshape(1, -1), be.reshape(1, -1)):
            flat.append(arr2)
            in_specs.append(pl.BlockSpec(arr2.shape, lambda i: (0, 0)))
    return pl.pallas_call(
        functools.partial(_sa_mlp_body, nlayers, cx, K, r2),
        grid=(Sp // Sb,),
        in_specs=in_specs,
        out_specs=pl.BlockSpec((Sb, cout), lambda i: (i, 0)),
        out_shape=jax.ShapeDtypeStruct((Sp, cout), jnp.float32),
    )(gath, pos_q, *flat)


def _fps_body(num, n_real, px_ref, py_ref, pz_ref, idx_ref, dists_ref):
    rows, cols = px_ref.shape
    flat = (jax.lax.broadcasted_iota(jnp.int32, (rows, cols), 0) * cols
            + jax.lax.broadcasted_iota(jnp.int32, (rows, cols), 1))
    real = flat < n_real
    px, py, pz = px_ref[...], py_ref[...], pz_ref[...]

    def extract(sel, arr):
        return jnp.sum(jnp.where(sel, arr, 0.0))

    sel0 = flat == 0
    ax, ay, az = extract(sel0, px), extract(sel0, py), extract(sel0, pz)
    idx_ref[0] = jnp.int32(0)
    dists_ref[...] = jnp.where(real, jnp.inf, -jnp.inf)

    def body(i, carry):
        ax, ay, az = carry
        d = (px - ax) ** 2 + (py - ay) ** 2 + (pz - az) ** 2
        dists = jnp.minimum(dists_ref[...], d)
        dists_ref[...] = dists
        m = jnp.max(dists)
        cand = jnp.where(dists == m, flat, jnp.int32(2 ** 30))
        nxt = jnp.min(cand)
        idx_ref[i] = nxt
        sel = flat == nxt
        return extract(sel, px), extract(sel, py), extract(sel, pz)

    jax.lax.fori_loop(1, num, body, (ax, ay, az), unroll=False)


def _thresh_body(K, q_ref, px_ref, py_ref, pz_ref, thr_ref):
    qx = q_ref[:, 0:1]
    qy = q_ref[:, 1:2]
    qz = q_ref[:, 2:3]
    dx = qx - px_ref[...]
    dy = qy - py_ref[...]
    dz = qz - pz_ref[...]
    d2 = dx * dx + dy * dy + dz * dz
    bits = jax.lax.bitcast_convert_type(d2, jnp.int32)
    prefix = jnp.zeros((q_ref.shape[0], 1), jnp.int32)
    for bit in range(30, -1, -1):
        trial = prefix | jnp.int32((1 << bit) - 1)
        cnt = jnp.sum((bits <= trial).astype(jnp.int32), axis=1, keepdims=True)
        prefix = jnp.where(cnt >= K, prefix, prefix | jnp.int32(1 << bit))
    thr = jax.lax.bitcast_convert_type(prefix, jnp.float32)
    thr_ref[...] = jnp.broadcast_to(thr, thr_ref.shape)


def _thresh(pos_q, px, py, pz, K=64):
    Sp = pos_q.shape[0]
    Np = px.shape[1]
    Sb = 128
    assert Sp % Sb == 0
    return pl.pallas_call(
        functools.partial(_thresh_body, K),
        grid=(Sp // Sb,),
        in_specs=[pl.BlockSpec((Sb, 3), lambda i: (i, 0))] + [
            pl.BlockSpec((1, Np), lambda i: (0, 0))] * 3,
        out_specs=pl.BlockSpec((Sb, 128), lambda i: (i, 0)),
        out_shape=jax.ShapeDtypeStruct((Sp, 128), jnp.float32),
    )(pos_q, px, py, pz)[:, 0]


def _fps(pos, num):
    N = pos.shape[0]
    cols = 128
    rows = (N + cols - 1) // cols
    rows = ((rows + 7) // 8) * 8
    npad = rows * cols - N
    p = jnp.pad(pos, ((0, npad), (0, 0)))
    px = p[:, 0].reshape(rows, cols)
    py = p[:, 1].reshape(rows, cols)
    pz = p[:, 2].reshape(rows, cols)
    return pl.pallas_call(
        functools.partial(_fps_body, num, N),
        in_specs=[pl.BlockSpec((rows, cols), lambda: (0, 0))] * 3,
        out_specs=pl.BlockSpec(memory_space=pltpu.SMEM),
        out_shape=jax.ShapeDtypeStruct((num,), jnp.int32),
        scratch_shapes=[pltpu.VMEM((rows, cols), jnp.float32)],
    )(px, py, pz)


def _select_sim(thr_b, pos_qp, posp, K=64):
    Np = posp.shape[0]
    dx = pos_qp[:, 0:1] - posp[None, :, 0]
    dy = pos_qp[:, 1:2] - posp[None, :, 1]
    dz = pos_qp[:, 2:3] - posp[None, :, 2]
    d2 = dx * dx + dy * dy + dz * dz
    mask = d2 <= thr_b[:, None]
    score = jnp.where(mask, jnp.arange(Np, dtype=jnp.int32), jnp.int32(Np))
    vals, _ = jax.lax.top_k(-score, K)
    idx = jnp.minimum(-vals, Np - 1)
    seld2 = jnp.take_along_axis(d2, idx, axis=1)
    return idx, seld2


def _gather_sim(table, idx, seld2, d2col):
    g = table[idx.reshape(-1)]
    return g.at[:, d2col].set(seld2.reshape(-1))


def _pad_rows(a, n, val=0.0):
    return jnp.pad(a, ((0, n - a.shape[0]), (0, 0)), constant_values=val)


def kernel(x, pos, batch, params):
    N = x.shape[0]
    s1 = int(N * 0.2)

    Np1, Sp1 = 10240, 2048
    posp1 = _pad_rows(pos, Np1, 1e6)
    px1 = posp1[:, 0].reshape(1, Np1)
    py1 = posp1[:, 1].reshape(1, Np1)
    pz1 = posp1[:, 2].reshape(1, Np1)
    idx1 = _fps(pos, s1)
    q1 = pos[idx1]
    q1p = _pad_rows(q1, Sp1)
    thr1 = _thresh(q1p, px1, py1, pz1) * np.float32(1.0 + 1e-6)
    table1 = jnp.pad(_pad_rows(jnp.concatenate([x, pos], axis=-1), Np1),
                     ((0, 0), (0, 10)))
    sel1, seld2_1 = _select_sim(thr1, q1p, posp1)
    g1 = _gather_sim(table1, sel1, seld2_1, 6)
    x1p = _sa_mlp(g1, q1p, params["mlp1"], np.float32(0.2 * 0.2), 128)
    x1 = x1p[:s1]
    pos1 = q1
    b1 = batch[idx1]

    s2 = int(s1 * 0.25)
    Np2, Sp2 = 2048, 512
    posp2 = _pad_rows(pos1, Np2, 1e6)
    px2 = posp2[:, 0].reshape(1, Np2)
    py2 = posp2[:, 1].reshape(1, Np2)
    pz2 = posp2[:, 2].reshape(1, Np2)
    idx2 = _fps(pos1, s2)
    q2 = pos1[idx2]
    q2p = _pad_rows(q2, Sp2)
    thr2 = _thresh(q2p, px2, py2, pz2) * np.float32(1.0 + 1e-6)
    table2 = jnp.pad(_pad_rows(jnp.concatenate([x1, pos1], axis=-1), Np2),
                     ((0, 0), (0, 13)))
    sel2, seld2_2 = _select_sim(thr2, q2p, posp2)
    g2 = _gather_sim(table2, sel2, seld2_2, 131)
    x2p = _sa_mlp(g2, q2p, params["mlp2"], np.float32(0.4 * 0.4), 64)
    x2 = x2p[:s2]
    pos2 = q2
    b2 = b1[idx2]
    h = jnp.concatenate([x2, pos2], axis=-1)
    hp = jnp.pad(h, ((0, 12), (0, 0)))
    hout = _mlp_pallas(hp, params["mlp3"], 512)[:500]
    x3 = jnp.max(hout, axis=0, keepdims=True)
    pos3 = jnp.zeros((1, 3), x.dtype)
    b3 = jnp.arange(1, dtype=jnp.int32)
    return ((x, pos, batch), (x1, pos1, b1), (x2, pos2, b2), (x3, pos3, b3))


# speedup vs baseline: 18.2388x; 5.7521x over previous
import functools

import jax
import jax.numpy as jnp
import numpy as np
from jax import lax
from jax.experimental import pallas as pl
from jax.experimental.pallas import tpu as pltpu
from jax.experimental.pallas import tpu_sc as plsc

_BN_SCALE = 1.0 / np.sqrt(1.0 + 1e-5)

_SC_NC, _SC_NS = 2, 16
_SC_NW = _SC_NC * _SC_NS


def _sc_select_body(Np, Sp, Dg, d2col, K,
                    px_h, py_h, pz_h, qx_h, qy_h, qz_h, thr_h, table_h,
                    gath_h,
                    pxv, pyv, pzv, qxv, qyv, qzv, thrv,
                    idxbuf, d2buf, idx64, rows_v, sem):
    wid = lax.axis_index("s") * _SC_NC + lax.axis_index("c")
    gpw = Sp // 16 // _SC_NW
    pltpu.sync_copy(px_h, pxv)
    pltpu.sync_copy(py_h, pyv)
    pltpu.sync_copy(pz_h, pzv)
    lanes = lax.broadcasted_iota(jnp.int32, (16,), 0)
    zeros16 = jnp.zeros((16,), jnp.int32)
    inf16 = jnp.full((16,), jnp.inf, jnp.float32)

    def group_body(gi, _):
        base = (wid * gpw + gi) * 16
        pltpu.sync_copy(qx_h.at[pl.ds(base, 16)], qxv)
        pltpu.sync_copy(qy_h.at[pl.ds(base, 16)], qyv)
        pltpu.sync_copy(qz_h.at[pl.ds(base, 16)], qzv)
        pltpu.sync_copy(thr_h.at[pl.ds(base, 16)], thrv)
        qx16, qy16, qz16, th16 = qxv[...], qyv[...], qzv[...], thrv[...]

        def row_body(si, _):
            lmf = (lanes == si).astype(jnp.float32)
            qxs = jnp.sum(qx16 * lmf)
            qys = jnp.sum(qy16 * lmf)
            qzs = jnp.sum(qz16 * lmf)
            ths = jnp.sum(th16 * lmf)
            for kk in range(K // 16 + 1):
                idxbuf[pl.ds(16 * kk, 16)] = zeros16
                d2buf[pl.ds(16 * kk, 16)] = inf16

            U = 8

            def scan_body(i, cnt):
                b0 = i * (16 * U)
                d2s, msks, cs = [], [], []
                for u in range(U):
                    o = b0 + u * 16
                    vx = pxv[pl.ds(o, 16)]
                    vy = pyv[pl.ds(o, 16)]
                    vz = pzv[pl.ds(o, 16)]
                    dx = vx - qxs
                    dy = vy - qys
                    dz = vz - qzs
                    d2u = dx * dx + dy * dy + dz * dz
                    mu = d2u <= ths
                    d2s.append(d2u)
                    msks.append(mu)
                    cs.append(plsc.all_reduce_population_count(mu))
                ctot_v = cs[0]
                for u in range(1, U):
                    ctot_v = ctot_v + cs[u]
                ctot = ctot_v[0]
                hit = jnp.logical_and(ctot > 0, cnt < K)

                def do_insert(c):
                    cl = c
                    for u in range(U):
                        plsc.store_compressed(idxbuf.at[pl.ds(cl, 16)],
                                              b0 + u * 16 + lanes,
                                              mask=msks[u])
                        plsc.store_compressed(d2buf.at[pl.ds(cl, 16)],
                                              d2s[u], mask=msks[u])
                        cl = cl + cs[u][0]
                    return c + ctot

                return lax.cond(hit, do_insert, lambda c: c, cnt)

            lax.fori_loop(0, Np // (16 * U), scan_body, jnp.int32(0))
            s = base + si
            for kk in range(K // 16):
                idx64[pl.ds(16 * kk, 16)] = idxbuf[pl.ds(16 * kk, 16)]
            pltpu.async_copy(table_h.at[idx64], rows_v, sem).wait()
            colv = jnp.full((16,), d2col, jnp.int32)
            for kk in range(K // 16):
                plsc.store_scatter(rows_v, [16 * kk + lanes, colv],
                                   d2buf[pl.ds(16 * kk, 16)])
            pltpu.sync_copy(rows_v, gath_h.at[pl.ds(s * K, K)])
            return 0

        lax.fori_loop(0, 16, row_body, 0)
        return 0

    lax.fori_loop(0, gpw, group_body, 0)


def _sc_select(posp, pos_qp, thr, table, d2col, K=64):
    Np = posp.shape[0]
    Sp = pos_qp.shape[0]
    Dg = table.shape[1]
    mesh = plsc.VectorSubcoreMesh(core_axis_name="c", subcore_axis_name="s",
                                  num_cores=_SC_NC, num_subcores=_SC_NS)
    f = pl.kernel(
        functools.partial(_sc_select_body, Np, Sp, Dg, d2col, K),
        out_type=jax.ShapeDtypeStruct((Sp * K, Dg), jnp.float32),
        mesh=mesh,
        scratch_types=[
            pltpu.VMEM((Np,), jnp.float32),
            pltpu.VMEM((Np,), jnp.float32),
            pltpu.VMEM((Np,), jnp.float32),
            pltpu.VMEM((16,), jnp.float32),
            pltpu.VMEM((16,), jnp.float32),
            pltpu.VMEM((16,), jnp.float32),
            pltpu.VMEM((16,), jnp.float32),
            pltpu.VMEM((K + 128,), jnp.int32),
            pltpu.VMEM((K + 128,), jnp.float32),
            pltpu.VMEM((K,), jnp.int32),
            pltpu.VMEM((K, Dg), jnp.float32),
            pltpu.SemaphoreType.DMA,
        ],
        compiler_params=pltpu.CompilerParams(use_tc_tiling_on_sc=False,
                                             needs_layout_passes=False),
    )
    return f(posp[:, 0], posp[:, 1], posp[:, 2],
             pos_qp[:, 0], pos_qp[:, 1], pos_qp[:, 2], thr, table)


def _mlp_body(nlayers, h_ref, *refs):
    o_ref = refs[-1]
    params = refs[:-1]
    h = h_ref[...]
    for li in range(nlayers):
        W, b, g, be = params[4 * li: 4 * li + 4]
        h = jax.lax.dot_general(h, W[...], (((1,), (0,)), ((), ())),
                                preferred_element_type=jnp.float32)
        h = h + b[...]
        h = jnp.maximum(h, 0.0)
        h = g[...] * (h * _BN_SCALE) + be[...]
    o_ref[...] = h


def _mlp_pallas(h, layers, block_rows):
    rows, cin = h.shape
    assert rows % block_rows == 0, (rows, block_rows)
    grid = rows // block_rows
    nlayers = len(layers)
    cout = layers[-1][0].shape[1]
    flat = []
    in_specs = [pl.BlockSpec((block_rows, cin), lambda i: (i, 0))]
    for (W, b, g, be) in layers:
        for arr2 in (W, b.reshape(1, -1), g.reshape(1, -1), be.reshape(1, -1)):
            flat.append(arr2)
            in_specs.append(pl.BlockSpec(arr2.shape, lambda i: (0, 0)))
    return pl.pallas_call(
        functools.partial(_mlp_body, nlayers),
        grid=(grid,),
        in_specs=in_specs,
        out_specs=pl.BlockSpec((block_rows, cout), lambda i: (i, 0)),
        out_shape=jax.ShapeDtypeStruct((rows, cout), jnp.float32),
    )(h, *flat)


def _sa_mlp_body(nlayers, cx, K, r2, g_ref, q_ref, *refs):
    o_ref = refs[-1]
    params = refs[:-1]
    Sb = q_ref.shape[0]
    xj = g_ref[:, :cx]
    pj = g_ref[:, cx:cx + 3]
    d2col = g_ref[:, cx + 3:cx + 4]
    rel = (pj.reshape(Sb, K, 3) - q_ref[...][:, None, :]).reshape(Sb * K, 3)
    h = jnp.concatenate([xj, rel], axis=-1)
    for li in range(nlayers):
        W, b, g, be = params[4 * li: 4 * li + 4]
        h = jax.lax.dot_general(h, W[...], (((1,), (0,)), ((), ())),
                                preferred_element_type=jnp.float32)
        h = h + b[...]
        h = jnp.maximum(h, 0.0)
        h = g[...] * (h * _BN_SCALE) + be[...]
    cout = h.shape[-1]
    h = jnp.where(d2col <= r2, h, -1e10)
    o_ref[...] = jnp.max(h.reshape(Sb, K, cout), axis=1)


def _sa_mlp(gath, pos_q, layers, r2, Sb, K=64):
    Sp = pos_q.shape[0]
    Dg = gath.shape[1]
    cx = layers[0][0].shape[0] - 3
    nlayers = len(layers)
    cout = layers[-1][0].shape[1]
    flat = []
    in_specs = [
        pl.BlockSpec((Sb * K, Dg), lambda i: (i, 0)),
        pl.BlockSpec((Sb, 3), lambda i: (i, 0)),
    ]
    for (W, b, g, be) in layers:
        for arr2 in (W, b.reshape(1, -1), g.reshape(1, -1), be.reshape(1, -1)):
            flat.append(arr2)
            in_specs.append(pl.BlockSpec(arr2.shape, lambda i: (0, 0)))
    return pl.pallas_call(
        functools.partial(_sa_mlp_body, nlayers, cx, K, r2),
        grid=(Sp // Sb,),
        in_specs=in_specs,
        out_specs=pl.BlockSpec((Sb, cout), lambda i: (i, 0)),
        out_shape=jax.ShapeDtypeStruct((Sp, cout), jnp.float32),
    )(gath, pos_q, *flat)


def _fps_body(num, n_real, px_ref, py_ref, pz_ref, psx, psy, psz,
              idx_ref, qx_ref, qy_ref, qz_ref, dists_ref):
    rows, cols = px_ref.shape
    flat = (jax.lax.broadcasted_iota(jnp.int32, (rows, cols), 0) * cols
            + jax.lax.broadcasted_iota(jnp.int32, (rows, cols), 1))
    real = flat < n_real
    px, py, pz = px_ref[...], py_ref[...], pz_ref[...]

    idx_ref[0] = jnp.int32(0)
    ax, ay, az = psx[0], psy[0], psz[0]
    qx_ref[0] = ax
    qy_ref[0] = ay
    qz_ref[0] = az
    dists_ref[...] = jnp.where(real, jnp.inf, -jnp.inf)

    def body(i, carry):
        ax, ay, az = carry
        d = (px - ax) ** 2 + (py - ay) ** 2 + (pz - az) ** 2
        dists = jnp.minimum(dists_ref[...], d)
        dists_ref[...] = dists
        m = jnp.max(dists)
        cand = jnp.where(dists == m, flat, jnp.int32(2 ** 30))
        nxt = jnp.min(cand)
        idx_ref[i] = nxt
        ax = psx[nxt]
        ay = psy[nxt]
        az = psz[nxt]
        qx_ref[i] = ax
        qy_ref[i] = ay
        qz_ref[i] = az
        return ax, ay, az

    jax.lax.fori_loop(1, num, body, (ax, ay, az), unroll=False)


def _thresh_body(K, q_ref, px_ref, py_ref, pz_ref, thr_ref):
    qx = q_ref[:, 0:1]
    qy = q_ref[:, 1:2]
    qz = q_ref[:, 2:3]
    dx = qx - px_ref[...]
    dy = qy - py_ref[...]
    dz = qz - pz_ref[...]
    d2 = dx * dx + dy * dy + dz * dz
    bits = jax.lax.bitcast_convert_type(d2, jnp.int32)
    prefix = jnp.zeros((q_ref.shape[0], 1), jnp.int32)
    for bit in range(30, -1, -1):
        trial = prefix | jnp.int32((1 << bit) - 1)
        cnt = jnp.sum((bits <= trial).astype(jnp.int32), axis=1, keepdims=True)
        prefix = jnp.where(cnt >= K, prefix, prefix | jnp.int32(1 << bit))
    thr = jax.lax.bitcast_convert_type(prefix, jnp.float32)
    thr_ref[...] = jnp.broadcast_to(thr, thr_ref.shape)


def _thresh(pos_q, px, py, pz, K=64):
    Sp = pos_q.shape[0]
    Np = px.shape[1]
    Sb = 128
    assert Sp % Sb == 0
    return pl.pallas_call(
        functools.partial(_thresh_body, K),
        grid=(Sp // Sb,),
        in_specs=[pl.BlockSpec((Sb, 3), lambda i: (i, 0))] + [
            pl.BlockSpec((1, Np), lambda i: (0, 0))] * 3,
        out_specs=pl.BlockSpec((Sb, 128), lambda i: (i, 0)),
        out_shape=jax.ShapeDtypeStruct((Sp, 128), jnp.float32),
    )(pos_q, px, py, pz)[:, 0]


def _fps(pos, num):
    N = pos.shape[0]
    cols = 128
    rows = (N + cols - 1) // cols
    rows = ((rows + 7) // 8) * 8
    npad = rows * cols - N
    p = jnp.pad(pos, ((0, npad), (0, 0)), constant_values=1e6)
    px = p[:, 0].reshape(rows, cols)
    py = p[:, 1].reshape(rows, cols)
    pz = p[:, 2].reshape(rows, cols)
    idx, qx, qy, qz = pl.pallas_call(
        functools.partial(_fps_body, num, N),
        in_specs=[pl.BlockSpec((rows, cols), lambda: (0, 0))] * 3
        + [pl.BlockSpec(memory_space=pltpu.SMEM)] * 3,
        out_specs=[pl.BlockSpec(memory_space=pltpu.SMEM)] * 4,
        out_shape=[jax.ShapeDtypeStruct((num,), jnp.int32)]
        + [jax.ShapeDtypeStruct((num,), jnp.float32)] * 3,
        scratch_shapes=[pltpu.VMEM((rows, cols), jnp.float32)],
    )(px, py, pz, p[:, 0], p[:, 1], p[:, 2])
    return idx, jnp.stack([qx, qy, qz], axis=1)


def _select_sim(thr_b, pos_qp, posp, K=64):
    Np = posp.shape[0]
    dx = pos_qp[:, 0:1] - posp[None, :, 0]
    dy = pos_qp[:, 1:2] - posp[None, :, 1]
    dz = pos_qp[:, 2:3] - posp[None, :, 2]
    d2 = dx * dx + dy * dy + dz * dz
    mask = d2 <= thr_b[:, None]
    score = jnp.where(mask, jnp.arange(Np, dtype=jnp.int32), jnp.int32(Np))
    vals, _ = jax.lax.top_k(-score, K)
    idx = jnp.minimum(-vals, Np - 1)
    seld2 = jnp.take_along_axis(d2, idx, axis=1)
    return idx, seld2


def _gather_sim(table, idx, seld2, d2col):
    g = table[idx.reshape(-1)]
    return g.at[:, d2col].set(seld2.reshape(-1))


def _pad_rows(a, n, val=0.0):
    return jnp.pad(a, ((0, n - a.shape[0]), (0, 0)), constant_values=val)


def kernel(x, pos, batch, params):
    N = x.shape[0]
    s1 = int(N * 0.2)

    Np1, Sp1 = 10240, 2048
    posp1 = _pad_rows(pos, Np1, 1e6)
    px1 = posp1[:, 0].reshape(1, Np1)
    py1 = posp1[:, 1].reshape(1, Np1)
    pz1 = posp1[:, 2].reshape(1, Np1)
    idx1, q1 = _fps(pos, s1)
    q1p = _pad_rows(q1, Sp1)
    thr1 = _thresh(q1p, px1, py1, pz1) * np.float32(1.0 + 1e-6)
    table1 = jnp.pad(_pad_rows(jnp.concatenate([x, pos], axis=-1), Np1),
                     ((0, 0), (0, 10)))
    g1 = _sc_select(posp1, q1p, thr1, table1, 6)
    x1p = _sa_mlp(g1, q1p, params["mlp1"], np.float32(0.2 * 0.2), 128)
    x1 = x1p[:s1]
    pos1 = q1
    b1 = batch[idx1]

    s2 = int(s1 * 0.25)
    Np2, Sp2 = 2048, 512
    posp2 = _pad_rows(pos1, Np2, 1e6)
    px2 = posp2[:, 0].reshape(1, Np2)
    py2 = posp2[:, 1].reshape(1, Np2)
    pz2 = posp2[:, 2].reshape(1, Np2)
    idx2, q2 = _fps(pos1, s2)
    q2p = _pad_rows(q2, Sp2)
    thr2 = _thresh(q2p, px2, py2, pz2) * np.float32(1.0 + 1e-6)
    table2 = jnp.pad(_pad_rows(jnp.concatenate([x1, pos1], axis=-1), Np2),
                     ((0, 0), (0, 13)))
    g2 = _sc_select(posp2, q2p, thr2, table2, 131)
    x2p = _sa_mlp(g2, q2p, params["mlp2"], np.float32(0.4 * 0.4), 64)
    x2 = x2p[:s2]
    pos2 = q2
    b2 = b1[idx2]
    h = jnp.concatenate([x2, pos2], axis=-1)
    hp = jnp.pad(h, ((0, 12), (0, 0)))
    hout = _mlp_pallas(hp, params["mlp3"], 512)[:500]
    x3 = jnp.max(hout, axis=0, keepdims=True)
    pos3 = jnp.zeros((1, 3), x.dtype)
    b3 = jnp.arange(1, dtype=jnp.int32)
    return ((x, pos, batch), (x1, pos1, b1), (x2, pos2, b2), (x3, pos3, b3))


# speedup vs baseline: 18.3498x; 1.0061x over previous
import functools

import jax
import jax.numpy as jnp
import numpy as np
from jax import lax
from jax.experimental import pallas as pl
from jax.experimental.pallas import tpu as pltpu
from jax.experimental.pallas import tpu_sc as plsc

_BN_SCALE = 1.0 / np.sqrt(1.0 + 1e-5)

_SC_NC, _SC_NS = 2, 16
_SC_NW = _SC_NC * _SC_NS


def _sc_select_body(Np, Sp, Dg, d2col, K,
                    px_h, py_h, pz_h, qx_h, qy_h, qz_h, thr_h, table_h,
                    gath_h,
                    pxv, pyv, pzv, qxv, qyv, qzv, thrv,
                    idxbuf, d2buf, idx64, rows_v, sem):
    wid = lax.axis_index("s") * _SC_NC + lax.axis_index("c")
    gpw = Sp // 16 // _SC_NW
    pltpu.sync_copy(px_h, pxv)
    pltpu.sync_copy(py_h, pyv)
    pltpu.sync_copy(pz_h, pzv)
    lanes = lax.broadcasted_iota(jnp.int32, (16,), 0)
    zeros16 = jnp.zeros((16,), jnp.int32)
    inf16 = jnp.full((16,), jnp.inf, jnp.float32)

    def group_body(gi, _):
        base = (wid * gpw + gi) * 16
        pltpu.sync_copy(qx_h.at[pl.ds(base, 16)], qxv)
        pltpu.sync_copy(qy_h.at[pl.ds(base, 16)], qyv)
        pltpu.sync_copy(qz_h.at[pl.ds(base, 16)], qzv)
        pltpu.sync_copy(thr_h.at[pl.ds(base, 16)], thrv)
        qx16, qy16, qz16, th16 = qxv[...], qyv[...], qzv[...], thrv[...]

        def row_body(si, _):
            lmf = (lanes == si).astype(jnp.float32)
            qxs = jnp.sum(qx16 * lmf)
            qys = jnp.sum(qy16 * lmf)
            qzs = jnp.sum(qz16 * lmf)
            ths = jnp.sum(th16 * lmf)
            for kk in range(K // 16 + 1):
                idxbuf[pl.ds(16 * kk, 16)] = zeros16
                d2buf[pl.ds(16 * kk, 16)] = inf16

            U = 8

            def scan_body(i, cnt):
                b0 = i * (16 * U)
                d2s, msks, cs = [], [], []
                for u in range(U):
                    o = b0 + u * 16
                    vx = pxv[pl.ds(o, 16)]
                    vy = pyv[pl.ds(o, 16)]
                    vz = pzv[pl.ds(o, 16)]
                    dx = vx - qxs
                    dy = vy - qys
                    dz = vz - qzs
                    d2u = dx * dx + dy * dy + dz * dz
                    mu = d2u <= ths
                    d2s.append(d2u)
                    msks.append(mu)
                    cs.append(plsc.all_reduce_population_count(mu))
                ctot_v = cs[0]
                for u in range(1, U):
                    ctot_v = ctot_v + cs[u]
                ctot = ctot_v[0]
                hit = jnp.logical_and(ctot > 0, cnt < K)

                def do_insert(c):
                    cl = c
                    for u in range(U):
                        plsc.store_compressed(idxbuf.at[pl.ds(cl, 16)],
                                              b0 + u * 16 + lanes,
                                              mask=msks[u])
                        plsc.store_compressed(d2buf.at[pl.ds(cl, 16)],
                                              d2s[u], mask=msks[u])
                        cl = cl + cs[u][0]
                    return c + ctot

                return lax.cond(hit, do_insert, lambda c: c, cnt)

            lax.fori_loop(0, Np // (16 * U), scan_body, jnp.int32(0))
            s = base + si
            for kk in range(K // 16):
                idx64[pl.ds(16 * kk, 16)] = idxbuf[pl.ds(16 * kk, 16)]
            pltpu.async_copy(table_h.at[idx64], rows_v, sem).wait()
            colv = jnp.full((16,), d2col, jnp.int32)
            for kk in range(K // 16):
                plsc.store_scatter(rows_v, [16 * kk + lanes, colv],
                                   d2buf[pl.ds(16 * kk, 16)])
            pltpu.sync_copy(rows_v, gath_h.at[pl.ds(s * K, K)])
            return 0

        lax.fori_loop(0, 16, row_body, 0)
        return 0

    lax.fori_loop(0, gpw, group_body, 0)


def _sc_select(posp, pos_qp, thr, table, d2col, K=64):
    Np = posp.shape[0]
    Sp = pos_qp.shape[0]
    Dg = table.shape[1]
    mesh = plsc.VectorSubcoreMesh(core_axis_name="c", subcore_axis_name="s",
                                  num_cores=_SC_NC, num_subcores=_SC_NS)
    f = pl.kernel(
        functools.partial(_sc_select_body, Np, Sp, Dg, d2col, K),
        out_type=jax.ShapeDtypeStruct((Sp * K, Dg), jnp.float32),
        mesh=mesh,
        scratch_types=[
            pltpu.VMEM((Np,), jnp.float32),
            pltpu.VMEM((Np,), jnp.float32),
            pltpu.VMEM((Np,), jnp.float32),
            pltpu.VMEM((16,), jnp.float32),
            pltpu.VMEM((16,), jnp.float32),
            pltpu.VMEM((16,), jnp.float32),
            pltpu.VMEM((16,), jnp.float32),
            pltpu.VMEM((K + 128,), jnp.int32),
            pltpu.VMEM((K + 128,), jnp.float32),
            pltpu.VMEM((K,), jnp.int32),
            pltpu.VMEM((K, Dg), jnp.float32),
            pltpu.SemaphoreType.DMA,
        ],
        compiler_params=pltpu.CompilerParams(use_tc_tiling_on_sc=False,
                                             needs_layout_passes=False),
    )
    return f(posp[:, 0], posp[:, 1], posp[:, 2],
             pos_qp[:, 0], pos_qp[:, 1], pos_qp[:, 2], thr, table)


def _mlp_body(nlayers, nreal, h_ref, *refs):
    o_ref = refs[-1]
    params = refs[:-1]
    h = h_ref[...]
    for li in range(nlayers):
        W, b, g, be = params[4 * li: 4 * li + 4]
        h = jax.lax.dot_general(h, W[...], (((1,), (0,)), ((), ())),
                                preferred_element_type=jnp.float32)
        h = h + b[...]
        h = jnp.maximum(h, 0.0)
        h = g[...] * (h * _BN_SCALE) + be[...]
    ri = jax.lax.broadcasted_iota(jnp.int32, (h.shape[0], 1), 0)
    h = jnp.where(ri < nreal, h, -1e10)
    m = jnp.max(h, axis=0, keepdims=True)
    o_ref[...] = jnp.broadcast_to(m, o_ref.shape)


def _mlp_pallas(h, layers, nreal):
    rows, cin = h.shape
    nlayers = len(layers)
    cout = layers[-1][0].shape[1]
    flat = []
    in_specs = [pl.BlockSpec((rows, cin), lambda: (0, 0))]
    for (W, b, g, be) in layers:
        for arr2 in (W, b.reshape(1, -1), g.reshape(1, -1), be.reshape(1, -1)):
            flat.append(arr2)
            in_specs.append(pl.BlockSpec(arr2.shape, lambda: (0, 0)))
    return pl.pallas_call(
        functools.partial(_mlp_body, nlayers, nreal),
        in_specs=in_specs,
        out_specs=pl.BlockSpec((8, cout), lambda: (0, 0)),
        out_shape=jax.ShapeDtypeStruct((8, cout), jnp.float32),
    )(h, *flat)


def _sa_mlp_body(nlayers, cx, K, r2, g_ref, q_ref, *refs):
    o_ref = refs[-1]
    params = refs[:-1]
    Sb = q_ref.shape[0]
    xj = g_ref[:, :cx]
    pj = g_ref[:, cx:cx + 3]
    d2col = g_ref[:, cx + 3:cx + 4]
    rel = (pj.reshape(Sb, K, 3) - q_ref[...][:, None, :]).reshape(Sb * K, 3)
    h = jnp.concatenate([xj, rel], axis=-1)
    for li in range(nlayers):
        W, b, g, be = params[4 * li: 4 * li + 4]
        h = jax.lax.dot_general(h, W[...], (((1,), (0,)), ((), ())),
                                preferred_element_type=jnp.float32)
        h = h + b[...]
        h = jnp.maximum(h, 0.0)
        h = g[...] * (h * _BN_SCALE) + be[...]
    cout = h.shape[-1]
    h = jnp.where(d2col <= r2, h, -1e10)
    o_ref[...] = jnp.max(h.reshape(Sb, K, cout), axis=1)


def _sa_mlp(gath, pos_q, layers, r2, Sb, K=64):
    Sp = pos_q.shape[0]
    Dg = gath.shape[1]
    cx = layers[0][0].shape[0] - 3
    nlayers = len(layers)
    cout = layers[-1][0].shape[1]
    flat = []
    in_specs = [
        pl.BlockSpec((Sb * K, Dg), lambda i: (i, 0)),
        pl.BlockSpec((Sb, 3), lambda i: (i, 0)),
    ]
    for (W, b, g, be) in layers:
        for arr2 in (W, b.reshape(1, -1), g.reshape(1, -1), be.reshape(1, -1)):
            flat.append(arr2)
            in_specs.append(pl.BlockSpec(arr2.shape, lambda i: (0, 0)))
    return pl.pallas_call(
        functools.partial(_sa_mlp_body, nlayers, cx, K, r2),
        grid=(Sp // Sb,),
        in_specs=in_specs,
        out_specs=pl.BlockSpec((Sb, cout), lambda i: (i, 0)),
        out_shape=jax.ShapeDtypeStruct((Sp, cout), jnp.float32),
    )(gath, pos_q, *flat)


def _fps_body(num, n_real, px_ref, py_ref, pz_ref, psx, psy, psz,
              idx_ref, qx_ref, qy_ref, qz_ref, dists_ref):
    rows, cols = px_ref.shape
    flat = (jax.lax.broadcasted_iota(jnp.int32, (rows, cols), 0) * cols
            + jax.lax.broadcasted_iota(jnp.int32, (rows, cols), 1))
    real = flat < n_real
    px, py, pz = px_ref[...], py_ref[...], pz_ref[...]

    idx_ref[0] = jnp.int32(0)
    ax, ay, az = psx[0], psy[0], psz[0]
    qx_ref[0] = ax
    qy_ref[0] = ay
    qz_ref[0] = az
    dists_ref[...] = jnp.where(real, jnp.inf, -jnp.inf)

    def body(i, carry):
        ax, ay, az = carry
        d = (px - ax) ** 2 + (py - ay) ** 2 + (pz - az) ** 2
        dists = jnp.minimum(dists_ref[...], d)
        dists_ref[...] = dists
        m = jnp.max(dists)
        cand = jnp.where(dists == m, flat, jnp.int32(2 ** 30))
        nxt = jnp.min(cand)
        idx_ref[i] = nxt
        ax = psx[nxt]
        ay = psy[nxt]
        az = psz[nxt]
        qx_ref[i] = ax
        qy_ref[i] = ay
        qz_ref[i] = az
        return ax, ay, az

    jax.lax.fori_loop(1, num, body, (ax, ay, az), unroll=False)


def _thresh_body(K, q_ref, px_ref, py_ref, pz_ref, thr_ref):
    qx = q_ref[:, 0:1]
    qy = q_ref[:, 1:2]
    qz = q_ref[:, 2:3]
    dx = qx - px_ref[...]
    dy = qy - py_ref[...]
    dz = qz - pz_ref[...]
    d2 = dx * dx + dy * dy + dz * dz
    bits = jax.lax.bitcast_convert_type(d2, jnp.int32)
    prefix = jnp.zeros((q_ref.shape[0], 1), jnp.int32)
    for bit in range(30, -1, -1):
        trial = prefix | jnp.int32((1 << bit) - 1)
        cnt = jnp.sum((bits <= trial).astype(jnp.int32), axis=1, keepdims=True)
        prefix = jnp.where(cnt >= K, prefix, prefix | jnp.int32(1 << bit))
    thr = jax.lax.bitcast_convert_type(prefix, jnp.float32)
    thr_ref[...] = jnp.broadcast_to(thr, thr_ref.shape)


def _thresh(pos_q, px, py, pz, K=64):
    Sp = pos_q.shape[0]
    Np = px.shape[1]
    Sb = 128
    assert Sp % Sb == 0
    return pl.pallas_call(
        functools.partial(_thresh_body, K),
        grid=(Sp // Sb,),
        in_specs=[pl.BlockSpec((Sb, 3), lambda i: (i, 0))] + [
            pl.BlockSpec((1, Np), lambda i: (0, 0))] * 3,
        out_specs=pl.BlockSpec((Sb, 128), lambda i: (i, 0)),
        out_shape=jax.ShapeDtypeStruct((Sp, 128), jnp.float32),
    )(pos_q, px, py, pz)[:, 0]


def _fps(pos, num):
    N = pos.shape[0]
    cols = 128
    rows = (N + cols - 1) // cols
    rows = ((rows + 7) // 8) * 8
    npad = rows * cols - N
    p = jnp.pad(pos, ((0, npad), (0, 0)), constant_values=1e6)
    px = p[:, 0].reshape(rows, cols)
    py = p[:, 1].reshape(rows, cols)
    pz = p[:, 2].reshape(rows, cols)
    idx, qx, qy, qz = pl.pallas_call(
        functools.partial(_fps_body, num, N),
        in_specs=[pl.BlockSpec((rows, cols), lambda: (0, 0))] * 3
        + [pl.BlockSpec(memory_space=pltpu.SMEM)] * 3,
        out_specs=[pl.BlockSpec(memory_space=pltpu.SMEM)] * 4,
        out_shape=[jax.ShapeDtypeStruct((num,), jnp.int32)]
        + [jax.ShapeDtypeStruct((num,), jnp.float32)] * 3,
        scratch_shapes=[pltpu.VMEM((rows, cols), jnp.float32)],
    )(px, py, pz, p[:, 0], p[:, 1], p[:, 2])
    return idx, jnp.stack([qx, qy, qz], axis=1)


def _pad_rows(a, n, val=0.0):
    return jnp.pad(a, ((0, n - a.shape[0]), (0, 0)), constant_values=val)


def kernel(x, pos, batch, params):
    N = x.shape[0]
    s1 = int(N * 0.2)

    Np1, Sp1 = 10240, 2048
    posp1 = _pad_rows(pos, Np1, 1e6)
    px1 = posp1[:, 0].reshape(1, Np1)
    py1 = posp1[:, 1].reshape(1, Np1)
    pz1 = posp1[:, 2].reshape(1, Np1)
    idx1, q1 = _fps(pos, s1)
    q1p = _pad_rows(q1, Sp1)
    thr1 = _thresh(q1p, px1, py1, pz1) * np.float32(1.0 + 1e-6)
    table1 = jnp.pad(_pad_rows(jnp.concatenate([x, pos], axis=-1), Np1),
                     ((0, 0), (0, 10)))
    g1 = _sc_select(posp1, q1p, thr1, table1, 6)
    x1p = _sa_mlp(g1, q1p, params["mlp1"], np.float32(0.2 * 0.2), 128)
    x1 = x1p[:s1]
    pos1 = q1
    b1 = batch[idx1]

    s2 = int(s1 * 0.25)
    Np2, Sp2 = 2048, 512
    posp2 = _pad_rows(pos1, Np2, 1e6)
    px2 = posp2[:, 0].reshape(1, Np2)
    py2 = posp2[:, 1].reshape(1, Np2)
    pz2 = posp2[:, 2].reshape(1, Np2)
    idx2, q2 = _fps(pos1, s2)
    q2p = _pad_rows(q2, Sp2)
    thr2 = _thresh(q2p, px2, py2, pz2) * np.float32(1.0 + 1e-6)
    table2 = jnp.pad(_pad_rows(jnp.concatenate([x1, pos1], axis=-1), Np2),
                     ((0, 0), (0, 13)))
    g2 = _sc_select(posp2, q2p, thr2, table2, 131)
    x2p = _sa_mlp(g2, q2p, params["mlp2"], np.float32(0.4 * 0.4), 64)
    x2 = x2p[:s2]
    pos2 = q2
    b2 = b1[idx2]
    h = jnp.concatenate([x2, pos2], axis=-1)
    hp = jnp.pad(h, ((0, 12), (0, 0)))
    x3 = _mlp_pallas(hp, params["mlp3"], s2)[:1]
    pos3 = jnp.zeros((1, 3), x.dtype)
    b3 = jnp.arange(1, dtype=jnp.int32)
    return ((x, pos, batch), (x1, pos1, b1), (x2, pos2, b2), (x3, pos3, b3))
